# baseline (speedup 1.0000x reference)
import functools

import jax
import jax.numpy as jnp
from jax import lax
from jax.experimental import pallas as pl
from jax.experimental.pallas import tpu as pltpu
from jax.experimental.pallas import tpu_sc as plsc

_N = 10000
_E = 320000
_F = 128
_H = 128
_C = 40
_G = 4
_LANES = 128
_NC = 2
_NS = 16
_NW = _NC * _NS
_EPB = 80
_NBP = _NW * _EPB
_EPAD = _NBP * _LANES
_NBUF = 2
_ROUNDS = _EPB // _NBUF
_CHB = 16
_NCHUNK = _EPB // _CHB
_NP = 10240
_RPS = _NP // _NS

_GRID_MIN, _GRID_MAX = -2.0, 2.0
_DENOM = (_GRID_MAX - _GRID_MIN) / (_G - 1)
_GRIDS = tuple(_GRID_MIN + i * _DENOM for i in range(_G))


def _sc_degree(col2d, ones_rows, zrows):
    mesh = plsc.VectorSubcoreMesh(core_axis_name="c", subcore_axis_name="s")

    @functools.partial(
        pl.kernel,
        mesh=mesh,
        out_type=jax.ShapeDtypeStruct((_NC, _NP, _H), jnp.float32),
        scratch_types=[
            pltpu.VMEM((_EPB, _LANES), jnp.int32),
            pltpu.VMEM((_LANES, _H), jnp.float32),
            pltpu.VMEM_SHARED((_NP, _H), jnp.float32),
            pltpu.SemaphoreType.DMA,
        ],
    )
    def run(col_hbm, ones_hbm, z_hbm, out_hbm, colb, onesv, acc, sem):
        c = lax.axis_index("c")
        s = lax.axis_index("s")
        wid = c * _NS + s
        base = pl.multiple_of(s * _RPS, _RPS)
        bbase = pl.multiple_of(wid * _EPB, 8)
        pltpu.sync_copy(z_hbm, acc.at[pl.ds(base, _RPS)])
        pltpu.sync_copy(col_hbm.at[pl.ds(bbase, _EPB)], colb)
        pltpu.sync_copy(ones_hbm, onesv)
        plsc.subcore_barrier()

        def round_body(r, carry):
            for j in range(8):
                k = r * 8 + j
                pltpu.async_copy(onesv, acc.at[colb.at[k]], sem, add=True)
            for j in range(8):
                k = r * 8 + j
                pltpu.make_async_copy(onesv, acc.at[colb.at[k]], sem).wait()
            return carry

        lax.fori_loop(0, _EPB // 8, round_body, 0)
        plsc.subcore_barrier()
        pltpu.sync_copy(acc.at[pl.ds(base, _RPS)],
                        out_hbm.at[c, pl.ds(base, _RPS)])

    return run(col2d, ones_rows, zrows)


def _sc_segment_sum(hs, row2d, col2d, zrows):
    mesh = plsc.VectorSubcoreMesh(core_axis_name="c", subcore_axis_name="s")

    @functools.partial(
        pl.kernel,
        mesh=mesh,
        out_type=jax.ShapeDtypeStruct((_NC, _NP, _H), jnp.float32),
        scratch_types=[
            pltpu.VMEM((_CHB, _LANES), jnp.int32),
            pltpu.VMEM((_CHB, _LANES), jnp.int32),
        ] + [pltpu.VMEM((_LANES, _H), jnp.float32)] * _NBUF
          + [pltpu.VMEM_SHARED((_NP, _H), jnp.float32)]
          + [pltpu.SemaphoreType.DMA] * _NBUF,
    )
    def run(hs_hbm, row_hbm, col_hbm, z_hbm, out_hbm, rowb, colb, *rest):
        rows = rest[:_NBUF]
        acc = rest[_NBUF]
        sems = rest[_NBUF + 1:]
        c = lax.axis_index("c")
        s = lax.axis_index("s")
        wid = c * _NS + s
        base = pl.multiple_of(s * _RPS, _RPS)
        bbase = pl.multiple_of(wid * _EPB, 8)
        pltpu.sync_copy(z_hbm, acc.at[pl.ds(base, _RPS)])
        plsc.subcore_barrier()

        def chunk_body(ci, carry):
            cb = pl.multiple_of(bbase + ci * _CHB, 8)
            pltpu.sync_copy(row_hbm.at[pl.ds(cb, _CHB)], rowb)
            pltpu.sync_copy(col_hbm.at[pl.ds(cb, _CHB)], colb)
            for j in range(_NBUF):
                pltpu.async_copy(hs_hbm.at[rowb.at[j]], rows[j], sems[j])
            for k in range(_NBUF, _CHB):
                j = k % _NBUF
                pltpu.make_async_copy(hs_hbm.at[rowb.at[k - _NBUF]],
                                      rows[j], sems[j]).wait()
                pltpu.sync_copy(rows[j], acc.at[colb.at[k - _NBUF]],
                                add=True)
                pltpu.async_copy(hs_hbm.at[rowb.at[k]], rows[j], sems[j])
            for k in range(_CHB - _NBUF, _CHB):
                j = k % _NBUF
                pltpu.make_async_copy(hs_hbm.at[rowb.at[k]], rows[j],
                                      sems[j]).wait()
                pltpu.sync_copy(rows[j], acc.at[colb.at[k]], add=True)
            return carry

        lax.fori_loop(0, _NCHUNK, chunk_body, 0)
        plsc.subcore_barrier()
        pltpu.sync_copy(acc.at[pl.ds(base, _RPS)],
                        out_hbm.at[c, pl.ds(base, _RPS)])

    return run(hs, row2d, col2d, zrows)


def _dis_from_parts(dp):
    deg = dp[0, :, 0:1] + dp[1, :, 0:1] + 1.0
    return lax.rsqrt(deg)


def _fastkan(x, ln_g, ln_b, swr, bwT, bb):
    m = jnp.mean(x, axis=1, keepdims=True)
    v = jnp.mean((x - m) ** 2, axis=1, keepdims=True)
    xn = (x - m) * lax.rsqrt(v + 1e-5) * ln_g + ln_b
    acc = jnp.dot(jax.nn.silu(x), bwT,
                  preferred_element_type=jnp.float32) + bb
    for g0 in range(_G):
        basis = jnp.exp(-(((xn - _GRIDS[g0]) / _DENOM) ** 2))
        acc = acc + jnp.dot(basis, swr[g0],
                            preferred_element_type=jnp.float32)
    return acc


_BLK = 2000
_NSTEP = _N // _BLK


def _full(shape):
    r = len(shape)
    return pl.BlockSpec(shape, lambda i, _r=r: (0,) * _r)


def _rows(shape):
    r = len(shape)
    return pl.BlockSpec((_BLK,) + tuple(shape[1:]),
                        lambda i, _r=r: (i,) + (0,) * (_r - 1))


_DP_SPEC = pl.BlockSpec((2, _BLK, _H), lambda i: (0, i, 0))


def _tc_stage1(x, dp, ln_g, ln_b, swr, bwT, bb):
    def body(x_ref, dp_ref, g_ref, b_ref, swr_ref, bwT_ref, bb_ref, hs_ref):
        dis = _dis_from_parts(dp_ref[...])
        h = _fastkan(x_ref[...], g_ref[...], b_ref[...], swr_ref[...],
                     bwT_ref[...], bb_ref[...])
        hs_ref[...] = h * dis

    return pl.pallas_call(
        body,
        grid=(_NSTEP,),
        in_specs=[_rows(x.shape), _DP_SPEC, _full(ln_g.shape),
                  _full(ln_b.shape), _full(swr.shape), _full(bwT.shape),
                  _full(bb.shape)],
        out_specs=_rows((_N, _H)),
        out_shape=jax.ShapeDtypeStruct((_N, _H), jnp.float32),
    )(x, dp, ln_g, ln_b, swr, bwT, bb)


def _tc_aggstats(p, hs, dp, bias):
    def body(p_ref, hs_ref, dp_ref, bias_ref, agg_ref, st_ref):
        i = pl.program_id(0)
        dis = _dis_from_parts(dp_ref[...])
        agg = (p_ref[0] + p_ref[1] + hs_ref[...]) * dis + bias_ref[...]
        agg_ref[...] = agg

        @pl.when(i == 0)
        def _():
            st_ref[...] = jnp.zeros((2, _H), jnp.float32)

        st_ref[...] += jnp.stack(
            [jnp.sum(agg, axis=0), jnp.sum(agg * agg, axis=0)])

    return pl.pallas_call(
        body,
        grid=(_NSTEP,),
        in_specs=[pl.BlockSpec((2, _BLK, _H), lambda i: (0, i, 0)),
                  _rows((_N, _H)), _DP_SPEC, _full(bias.shape)],
        out_specs=[_rows((_N, _H)), _full((2, _H))],
        out_shape=[jax.ShapeDtypeStruct((_N, _H), jnp.float32),
                   jax.ShapeDtypeStruct((2, _H), jnp.float32)],
    )(p, hs, dp, bias)


def _bn_from_stats(x, st, g, b):
    m = st[0:1] / float(_N)
    v = st[1:2] / float(_N) - m * m
    return (x - m) * lax.rsqrt(v + 1e-5) * g + b


def _tc_stage2(agg, st, dp, bn_g, bn_b, ln_g, ln_b, swr, bwT, bb):
    def body(agg_ref, st_ref, dp_ref, bng_ref, bnb_ref, lng_ref, lnb_ref,
             swr_ref, bwT_ref, bb_ref, h1p_ref, hs2_ref):
        dis = _dis_from_parts(dp_ref[...])
        h1p = _bn_from_stats(agg_ref[...], st_ref[...], bng_ref[...],
                             bnb_ref[...])
        h1p_ref[...] = h1p
        h2 = _fastkan(h1p, lng_ref[...], lnb_ref[...], swr_ref[...],
                      bwT_ref[...], bb_ref[...])
        hs2_ref[...] = h2 * dis

    return pl.pallas_call(
        body,
        grid=(_NSTEP,),
        in_specs=[_rows((_N, _H)), _full((2, _H)), _DP_SPEC,
                  _full(bn_g.shape), _full(bn_b.shape), _full(ln_g.shape),
                  _full(ln_b.shape), _full(swr.shape), _full(bwT.shape),
                  _full(bb.shape)],
        out_specs=[_rows((_N, _H)), _rows((_N, _H))],
        out_shape=[jax.ShapeDtypeStruct((_N, _H), jnp.float32),
                   jax.ShapeDtypeStruct((_N, _H), jnp.float32)],
    )(agg, st, dp, bn_g, bn_b, ln_g, ln_b, swr, bwT, bb)


def _tc_stage3(x, h1p, agg2, st2, bn_g, bn_b, lng_r, lnb_r, swro, bwTo, bbo):
    def body(x_ref, h1p_ref, agg_ref, st_ref, bng_ref, bnb_ref, lng_ref,
             lnb_ref, swr_ref, bwT_ref, bb_ref, out_ref):
        h2p = _bn_from_stats(agg_ref[...], st_ref[...], bng_ref[...],
                             bnb_ref[...])
        pieces = (x_ref[...], h1p_ref[...], h2p)
        din = float(3 * _H)
        m = (sum(jnp.sum(p, axis=1, keepdims=True) for p in pieces)) / din
        ssd = sum(jnp.sum((p - m) ** 2, axis=1, keepdims=True)
                  for p in pieces)
        inv = lax.rsqrt(ssd / din + 1e-5)
        acc = jnp.zeros((_BLK, _C), jnp.float32) + bb_ref[...]
        for pi, piece in enumerate(pieces):
            xn = (piece - m) * inv * lng_ref[pi] + lnb_ref[pi]
            acc = acc + jnp.dot(jax.nn.silu(piece), bwT_ref[pi],
                                preferred_element_type=jnp.float32)
            for g0 in range(_G):
                basis = jnp.exp(-(((xn - _GRIDS[g0]) / _DENOM) ** 2))
                acc = acc + jnp.dot(basis, swr_ref[pi, g0],
                                    preferred_element_type=jnp.float32)
        out_ref[...] = acc

    return pl.pallas_call(
        body,
        grid=(_NSTEP,),
        in_specs=[_rows((_N, _F)), _rows((_N, _H)), _rows((_N, _H)),
                  _full((2, _H)), _full(bn_g.shape), _full(bn_b.shape),
                  _full(lng_r.shape), _full(lnb_r.shape), _full(swro.shape),
                  _full(bwTo.shape), _full(bbo.shape)],
        out_specs=_rows((_N, _C)),
        out_shape=jax.ShapeDtypeStruct((_N, _C), jnp.float32),
    )(x, h1p, agg2, st2, bn_g, bn_b, lng_r, lnb_r, swro, bwTo, bbo)


def kernel(x, edge_index, ln_g1, ln_b1, sw1, bw1, bb1, bias1, bn_g1, bn_b1,
           ln_g2, ln_b2, sw2, bw2, bb2, bias2, bn_g2, bn_b2, ln_go, ln_bo,
           swo, bwo, bbo):
    pad = _EPAD - _E
    row2d = jnp.concatenate(
        [edge_index[0], jnp.zeros((pad,), jnp.int32)]).reshape(_NBP, _LANES)
    col2d = jnp.concatenate(
        [edge_index[1], jnp.full((pad,), _N, jnp.int32)]).reshape(_NBP,
                                                                  _LANES)
    zrows = jnp.zeros((_RPS, _H), jnp.float32)
    ones_rows = jnp.ones((_LANES, _H), jnp.float32)

    swr1 = jnp.transpose(sw1.reshape(_H, _F, _G), (2, 1, 0))
    swr2 = jnp.transpose(sw2.reshape(_H, _H, _G), (2, 1, 0))
    swro = jnp.transpose(swo.reshape(_C, 3, _H, _G), (1, 3, 2, 0))
    bwTo = jnp.transpose(bwo.reshape(_C, 3, _H), (1, 2, 0))

    dp = _sc_degree(col2d, ones_rows, zrows)[:, :_N]
    hs1 = _tc_stage1(x, dp, ln_g1, ln_b1, swr1, bw1.T, bb1)
    p1 = _sc_segment_sum(hs1, row2d, col2d, zrows)[:, :_N]
    agg1, st1 = _tc_aggstats(p1, hs1, dp, bias1)
    h1p, hs2 = _tc_stage2(agg1, st1, dp, bn_g1, bn_b1, ln_g2, ln_b2,
                          swr2, bw2.T, bb2)
    p2 = _sc_segment_sum(hs2, row2d, col2d, zrows)[:, :_N]
    agg2, st2 = _tc_aggstats(p2, hs2, dp, bias2)
    return _tc_stage3(x, h1p, agg2, st2, bn_g2, bn_b2,
                      ln_go.reshape(3, _H), ln_bo.reshape(3, _H), swro,
                      bwTo, bbo)


# speedup vs baseline: 2.6058x; 2.6058x over previous
import functools

import jax
import jax.numpy as jnp
from jax import lax
from jax.experimental import pallas as pl
from jax.experimental.pallas import tpu as pltpu
from jax.experimental.pallas import tpu_sc as plsc

_N = 10000
_E = 320000
_F = 128
_H = 128
_C = 40
_G = 4
_LANES = 128
_NC = 2
_NS = 16
_NW = _NC * _NS
_EPB = 80
_NBP = _NW * _EPB
_EPAD = _NBP * _LANES
_NBUF = 2
_ROUNDS = _EPB // _NBUF
_CHB = 16
_NCHUNK = _EPB // _CHB
_NP = 10240
_RPS = _NP // _NS

_GRID_MIN, _GRID_MAX = -2.0, 2.0
_DENOM = (_GRID_MAX - _GRID_MIN) / (_G - 1)
_GRIDS = tuple(_GRID_MIN + i * _DENOM for i in range(_G))


def _sc_degree(col2d, ones_rows, zrows):
    mesh = plsc.VectorSubcoreMesh(core_axis_name="c", subcore_axis_name="s")

    @functools.partial(
        pl.kernel,
        mesh=mesh,
        out_type=jax.ShapeDtypeStruct((_NC, _NP, _H), jnp.float32),
        scratch_types=[
            pltpu.VMEM((_EPB, _LANES), jnp.int32),
            pltpu.VMEM((_LANES, _H), jnp.float32),
            pltpu.VMEM_SHARED((_NP, _H), jnp.float32),
            pltpu.SemaphoreType.DMA,
        ],
    )
    def run(col_hbm, ones_hbm, z_hbm, out_hbm, colb, onesv, acc, sem):
        c = lax.axis_index("c")
        s = lax.axis_index("s")
        wid = c * _NS + s
        base = pl.multiple_of(s * _RPS, _RPS)
        bbase = pl.multiple_of(wid * _EPB, 8)
        pltpu.sync_copy(z_hbm, acc.at[pl.ds(base, _RPS)])
        pltpu.sync_copy(col_hbm.at[pl.ds(bbase, _EPB)], colb)
        pltpu.sync_copy(ones_hbm, onesv)
        plsc.subcore_barrier()

        def round_body(r, carry):
            for j in range(8):
                k = r * 8 + j
                pltpu.async_copy(onesv, acc.at[colb.at[k]], sem, add=True)
            for j in range(8):
                k = r * 8 + j
                pltpu.make_async_copy(onesv, acc.at[colb.at[k]], sem).wait()
            return carry

        lax.fori_loop(0, _EPB // 8, round_body, 0)
        plsc.subcore_barrier()
        pltpu.sync_copy(acc.at[pl.ds(base, _RPS)],
                        out_hbm.at[c, pl.ds(base, _RPS)])

    return run(col2d, ones_rows, zrows)


def _sc_segment_sum(hs, row2d, col2d, zrows):
    mesh = plsc.VectorSubcoreMesh(core_axis_name="c", subcore_axis_name="s")

    @functools.partial(
        pl.kernel,
        mesh=mesh,
        out_type=jax.ShapeDtypeStruct((_NC, _NP, _H), jnp.float32),
        scratch_types=[
            pltpu.VMEM((_CHB, _LANES), jnp.int32),
            pltpu.VMEM((_CHB, _LANES), jnp.int32),
        ] + [pltpu.VMEM((_LANES, _H), jnp.float32)] * _NBUF
          + [pltpu.VMEM_SHARED((_NP, _H), jnp.float32)]
          + [pltpu.SemaphoreType.DMA] * _NBUF,
    )
    def run(hs_hbm, row_hbm, col_hbm, z_hbm, out_hbm, rowb, colb, *rest):
        rows = rest[:_NBUF]
        acc = rest[_NBUF]
        sems = rest[_NBUF + 1:]
        c = lax.axis_index("c")
        s = lax.axis_index("s")
        wid = c * _NS + s
        base = pl.multiple_of(s * _RPS, _RPS)
        bbase = pl.multiple_of(wid * _EPB, 8)
        pltpu.sync_copy(z_hbm, acc.at[pl.ds(base, _RPS)])
        plsc.subcore_barrier()

        def chunk_body(ci, carry):
            cb = pl.multiple_of(bbase + ci * _CHB, 8)
            pltpu.sync_copy(row_hbm.at[pl.ds(cb, _CHB)], rowb)
            pltpu.sync_copy(col_hbm.at[pl.ds(cb, _CHB)], colb)
            for j in range(_NBUF):
                pltpu.async_copy(hs_hbm.at[rowb.at[j]], rows[j], sems[j])
            for k in range(_NBUF, _CHB):
                j = k % _NBUF
                pltpu.make_async_copy(hs_hbm.at[rowb.at[k - _NBUF]],
                                      rows[j], sems[j]).wait()
                pltpu.sync_copy(rows[j], acc.at[colb.at[k - _NBUF]],
                                add=True)
                pltpu.async_copy(hs_hbm.at[rowb.at[k]], rows[j], sems[j])
            for k in range(_CHB - _NBUF, _CHB):
                j = k % _NBUF
                pltpu.make_async_copy(hs_hbm.at[rowb.at[k]], rows[j],
                                      sems[j]).wait()
                pltpu.sync_copy(rows[j], acc.at[colb.at[k]], add=True)
            return carry

        lax.fori_loop(0, _NCHUNK, chunk_body, 0)
        plsc.subcore_barrier()
        pltpu.sync_copy(acc.at[pl.ds(base, _RPS)],
                        out_hbm.at[c, pl.ds(base, _RPS)])

    return run(hs, row2d, col2d, zrows)


def _dis_from_parts(dp):
    deg = dp[0, :, 0:1] + dp[1, :, 0:1] + 1.0
    return lax.rsqrt(deg)


def _fastkan(x, ln_g, ln_b, swr, bwT, bb):
    m = jnp.mean(x, axis=1, keepdims=True)
    v = jnp.mean((x - m) ** 2, axis=1, keepdims=True)
    xn = (x - m) * lax.rsqrt(v + 1e-5) * ln_g + ln_b
    acc = jnp.dot(jax.nn.silu(x), bwT,
                  preferred_element_type=jnp.float32) + bb
    for g0 in range(_G):
        basis = jnp.exp(-(((xn - _GRIDS[g0]) / _DENOM) ** 2))
        acc = acc + jnp.dot(basis, swr[g0],
                            preferred_element_type=jnp.float32)
    return acc


_BLK = 2000
_NSTEP = _N // _BLK


def _full(shape):
    r = len(shape)
    return pl.BlockSpec(shape, lambda i, _r=r: (0,) * _r)


def _rows(shape):
    r = len(shape)
    return pl.BlockSpec((_BLK,) + tuple(shape[1:]),
                        lambda i, _r=r: (i,) + (0,) * (_r - 1))


_DP_SPEC = pl.BlockSpec((2, _BLK, _H), lambda i: (0, i, 0))


def _tc_stage1(x, dp, ln_g, ln_b, swr, bwT, bb):
    def body(x_ref, dp_ref, g_ref, b_ref, swr_ref, bwT_ref, bb_ref, hs_ref):
        dis = _dis_from_parts(dp_ref[...])
        h = _fastkan(x_ref[...], g_ref[...], b_ref[...], swr_ref[...],
                     bwT_ref[...], bb_ref[...])
        hs_ref[...] = h * dis

    return pl.pallas_call(
        body,
        grid=(_NSTEP,),
        in_specs=[_rows(x.shape), _DP_SPEC, _full(ln_g.shape),
                  _full(ln_b.shape), _full(swr.shape), _full(bwT.shape),
                  _full(bb.shape)],
        out_specs=_rows((_N, _H)),
        out_shape=jax.ShapeDtypeStruct((_N, _H), jnp.float32),
    )(x, dp, ln_g, ln_b, swr, bwT, bb)


def _tc_aggstats(p, hs, dp, bias):
    def body(p_ref, hs_ref, dp_ref, bias_ref, agg_ref, st_ref):
        i = pl.program_id(0)
        dis = _dis_from_parts(dp_ref[...])
        agg = (p_ref[0] + p_ref[1] + hs_ref[...]) * dis + bias_ref[...]
        agg_ref[...] = agg

        @pl.when(i == 0)
        def _():
            st_ref[...] = jnp.zeros((2, _H), jnp.float32)

        st_ref[...] += jnp.stack(
            [jnp.sum(agg, axis=0), jnp.sum(agg * agg, axis=0)])

    return pl.pallas_call(
        body,
        grid=(_NSTEP,),
        in_specs=[pl.BlockSpec((2, _BLK, _H), lambda i: (0, i, 0)),
                  _rows((_N, _H)), _DP_SPEC, _full(bias.shape)],
        out_specs=[_rows((_N, _H)), _full((2, _H))],
        out_shape=[jax.ShapeDtypeStruct((_N, _H), jnp.float32),
                   jax.ShapeDtypeStruct((2, _H), jnp.float32)],
    )(p, hs, dp, bias)


def _bn_from_stats(x, st, g, b):
    m = st[0:1] / float(_N)
    v = st[1:2] / float(_N) - m * m
    return (x - m) * lax.rsqrt(v + 1e-5) * g + b


def _tc_stage2(agg, st, dp, bn_g, bn_b, ln_g, ln_b, swr, bwT, bb):
    def body(agg_ref, st_ref, dp_ref, bng_ref, bnb_ref, lng_ref, lnb_ref,
             swr_ref, bwT_ref, bb_ref, h1p_ref, hs2_ref):
        dis = _dis_from_parts(dp_ref[...])
        h1p = _bn_from_stats(agg_ref[...], st_ref[...], bng_ref[...],
                             bnb_ref[...])
        h1p_ref[...] = h1p
        h2 = _fastkan(h1p, lng_ref[...], lnb_ref[...], swr_ref[...],
                      bwT_ref[...], bb_ref[...])
        hs2_ref[...] = h2 * dis

    return pl.pallas_call(
        body,
        grid=(_NSTEP,),
        in_specs=[_rows((_N, _H)), _full((2, _H)), _DP_SPEC,
                  _full(bn_g.shape), _full(bn_b.shape), _full(ln_g.shape),
                  _full(ln_b.shape), _full(swr.shape), _full(bwT.shape),
                  _full(bb.shape)],
        out_specs=[_rows((_N, _H)), _rows((_N, _H))],
        out_shape=[jax.ShapeDtypeStruct((_N, _H), jnp.float32),
                   jax.ShapeDtypeStruct((_N, _H), jnp.float32)],
    )(agg, st, dp, bn_g, bn_b, ln_g, ln_b, swr, bwT, bb)


def _tc_stage3(x, h1p, agg2, st2, bn_g, bn_b, lng_r, lnb_r, swro, bwTo, bbo):
    def body(x_ref, h1p_ref, agg_ref, st_ref, bng_ref, bnb_ref, lng_ref,
             lnb_ref, swr_ref, bwT_ref, bb_ref, out_ref):
        h2p = _bn_from_stats(agg_ref[...], st_ref[...], bng_ref[...],
                             bnb_ref[...])
        pieces = (x_ref[...], h1p_ref[...], h2p)
        din = float(3 * _H)
        m = (sum(jnp.sum(p, axis=1, keepdims=True) for p in pieces)) / din
        ssd = sum(jnp.sum((p - m) ** 2, axis=1, keepdims=True)
                  for p in pieces)
        inv = lax.rsqrt(ssd / din + 1e-5)
        acc = jnp.zeros((_BLK, _C), jnp.float32) + bb_ref[...]
        for pi, piece in enumerate(pieces):
            xn = (piece - m) * inv * lng_ref[pi] + lnb_ref[pi]
            acc = acc + jnp.dot(jax.nn.silu(piece), bwT_ref[pi],
                                preferred_element_type=jnp.float32)
            for g0 in range(_G):
                basis = jnp.exp(-(((xn - _GRIDS[g0]) / _DENOM) ** 2))
                acc = acc + jnp.dot(basis, swr_ref[pi, g0],
                                    preferred_element_type=jnp.float32)
        out_ref[...] = acc

    return pl.pallas_call(
        body,
        grid=(_NSTEP,),
        in_specs=[_rows((_N, _F)), _rows((_N, _H)), _rows((_N, _H)),
                  _full((2, _H)), _full(bn_g.shape), _full(bn_b.shape),
                  _full(lng_r.shape), _full(lnb_r.shape), _full(swro.shape),
                  _full(bwTo.shape), _full(bbo.shape)],
        out_specs=_rows((_N, _C)),
        out_shape=jax.ShapeDtypeStruct((_N, _C), jnp.float32),
    )(x, h1p, agg2, st2, bn_g, bn_b, lng_r, lnb_r, swro, bwTo, bbo)


def kernel(x, edge_index, ln_g1, ln_b1, sw1, bw1, bb1, bias1, bn_g1, bn_b1,
           ln_g2, ln_b2, sw2, bw2, bb2, bias2, bn_g2, bn_b2, ln_go, ln_bo,
           swo, bwo, bbo):
    pad = _EPAD - _E
    pad_iota = lax.iota(jnp.int32, pad)
    row2d = jnp.concatenate(
        [edge_index[0], pad_iota % _N]).reshape(_NBP, _LANES)
    col2d = jnp.concatenate(
        [edge_index[1], _N + pad_iota % (_NP - _N)]).reshape(_NBP, _LANES)
    zrows = jnp.zeros((_RPS, _H), jnp.float32)
    ones_rows = jnp.ones((_LANES, _H), jnp.float32)

    swr1 = jnp.transpose(sw1.reshape(_H, _F, _G), (2, 1, 0))
    swr2 = jnp.transpose(sw2.reshape(_H, _H, _G), (2, 1, 0))
    swro = jnp.transpose(swo.reshape(_C, 3, _H, _G), (1, 3, 2, 0))
    bwTo = jnp.transpose(bwo.reshape(_C, 3, _H), (1, 2, 0))

    dp = _sc_degree(col2d, ones_rows, zrows)[:, :_N]
    hs1 = _tc_stage1(x, dp, ln_g1, ln_b1, swr1, bw1.T, bb1)
    p1 = _sc_segment_sum(hs1, row2d, col2d, zrows)[:, :_N]
    agg1, st1 = _tc_aggstats(p1, hs1, dp, bias1)
    h1p, hs2 = _tc_stage2(agg1, st1, dp, bn_g1, bn_b1, ln_g2, ln_b2,
                          swr2, bw2.T, bb2)
    p2 = _sc_segment_sum(hs2, row2d, col2d, zrows)[:, :_N]
    agg2, st2 = _tc_aggstats(p2, hs2, dp, bias2)
    return _tc_stage3(x, h1p, agg2, st2, bn_g2, bn_b2,
                      ln_go.reshape(3, _H), ln_bo.reshape(3, _H), swro,
                      bwTo, bbo)


# speedup vs baseline: 2.7602x; 1.0592x over previous
import functools

import jax
import jax.numpy as jnp
from jax import lax
from jax.experimental import pallas as pl
from jax.experimental.pallas import tpu as pltpu
from jax.experimental.pallas import tpu_sc as plsc

_N = 10000
_E = 320000
_F = 128
_H = 128
_C = 40
_G = 4
_LANES = 128
_NC = 2
_NS = 16
_NW = _NC * _NS
_EPB = 80
_NBP = _NW * _EPB
_EPAD = _NBP * _LANES
_GL = 64
_GEPB = _EPAD // (_NW * _GL)
_GCHB = 32
_GNCH = _GEPB // _GCHB
_GNBUF = 4
_NP = 10240
_RPS = _NP // _NS

_GRID_MIN, _GRID_MAX = -2.0, 2.0
_DENOM = (_GRID_MAX - _GRID_MIN) / (_G - 1)
_GRIDS = tuple(_GRID_MIN + i * _DENOM for i in range(_G))


def _sc_degree(col2d, ones_rows, zrows):
    mesh = plsc.VectorSubcoreMesh(core_axis_name="c", subcore_axis_name="s")

    @functools.partial(
        pl.kernel,
        mesh=mesh,
        out_type=jax.ShapeDtypeStruct((_NC, _NP, _H), jnp.float32),
        scratch_types=[
            pltpu.VMEM((_EPB, _LANES), jnp.int32),
            pltpu.VMEM((_LANES, _H), jnp.float32),
            pltpu.VMEM_SHARED((_NP, _H), jnp.float32),
            pltpu.SemaphoreType.DMA,
        ],
    )
    def run(col_hbm, ones_hbm, z_hbm, out_hbm, colb, onesv, acc, sem):
        c = lax.axis_index("c")
        s = lax.axis_index("s")
        wid = c * _NS + s
        base = pl.multiple_of(s * _RPS, _RPS)
        bbase = pl.multiple_of(wid * _EPB, 8)
        pltpu.sync_copy(z_hbm, acc.at[pl.ds(base, _RPS)])
        pltpu.sync_copy(col_hbm.at[pl.ds(bbase, _EPB)], colb)
        pltpu.sync_copy(ones_hbm, onesv)
        plsc.subcore_barrier()

        def round_body(r, carry):
            for j in range(8):
                k = r * 8 + j
                pltpu.async_copy(onesv, acc.at[colb.at[k]], sem, add=True)
            for j in range(8):
                k = r * 8 + j
                pltpu.make_async_copy(onesv, acc.at[colb.at[k]], sem).wait()
            return carry

        lax.fori_loop(0, _EPB // 8, round_body, 0)
        plsc.subcore_barrier()
        pltpu.sync_copy(acc.at[pl.ds(base, _RPS)],
                        out_hbm.at[c, pl.ds(base, _RPS)])

    return run(col2d, ones_rows, zrows)


def _sc_segment_sum(hs, row2d, col2d, zrows):
    mesh = plsc.VectorSubcoreMesh(core_axis_name="c", subcore_axis_name="s")

    @functools.partial(
        pl.kernel,
        mesh=mesh,
        out_type=jax.ShapeDtypeStruct((_NC, _NP, _H), jnp.float32),
        scratch_types=(
            [pltpu.VMEM((_GCHB, _GL), jnp.int32)] * 4
            + [pltpu.VMEM((_GL, _H), jnp.float32)] * _GNBUF
            + [pltpu.VMEM_SHARED((_NP, _H), jnp.float32)]
            + [pltpu.SemaphoreType.DMA] * (2 * _GNBUF + 4)
        ),
    )
    def run(hs_hbm, row_hbm, col_hbm, z_hbm, out_hbm, *rest):
        rowb = rest[0:2]
        colb = rest[2:4]
        rows = rest[4:4 + _GNBUF]
        acc = rest[4 + _GNBUF]
        sems = rest[5 + _GNBUF:]
        gsem = sems[0:_GNBUF]
        ssem = sems[_GNBUF:2 * _GNBUF]
        ibr = sems[2 * _GNBUF:2 * _GNBUF + 2]
        ibc = sems[2 * _GNBUF + 2:2 * _GNBUF + 4]
        c = lax.axis_index("c")
        s = lax.axis_index("s")
        wid = c * _NS + s
        base = pl.multiple_of(s * _RPS, _RPS)
        bbase = pl.multiple_of(wid * _GEPB, 8)

        def idx_fetch(ch):
            pb = ch % 2
            cb = pl.multiple_of(bbase + ch * _GCHB, 8)
            pltpu.async_copy(row_hbm.at[pl.ds(cb, _GCHB)], rowb[pb], ibr[pb])
            pltpu.async_copy(col_hbm.at[pl.ds(cb, _GCHB)], colb[pb], ibc[pb])

        def idx_wait(ch):
            pb = ch % 2
            cb = pl.multiple_of(bbase + ch * _GCHB, 8)
            pltpu.make_async_copy(row_hbm.at[pl.ds(cb, _GCHB)], rowb[pb],
                                  ibr[pb]).wait()
            pltpu.make_async_copy(col_hbm.at[pl.ds(cb, _GCHB)], colb[pb],
                                  ibc[pb]).wait()

        pltpu.sync_copy(z_hbm, acc.at[pl.ds(base, _RPS)])
        idx_fetch(0)
        idx_fetch(1)
        plsc.subcore_barrier()

        for t in range(_GEPB + 2):
            if t < _GEPB:
                k = t
                ch, off = divmod(k, _GCHB)
                pb = ch % 2
                j = k % _GNBUF
                if off == 0:
                    idx_wait(ch)
                if off == 4 and ch + 1 < _GNCH:
                    idx_fetch(ch + 1)
                if k >= _GNBUF:
                    kp = k - _GNBUF
                    pbp = (kp // _GCHB) % 2
                    pltpu.make_async_copy(
                        rows[j], acc.at[colb[pbp].at[kp % _GCHB]],
                        ssem[j]).wait()
                pltpu.async_copy(hs_hbm.at[rowb[pb].at[off]], rows[j],
                                 gsem[j])
            if t >= 2:
                k2 = t - 2
                ch2, off2 = divmod(k2, _GCHB)
                pb2 = ch2 % 2
                j2 = k2 % _GNBUF
                pltpu.make_async_copy(hs_hbm.at[rowb[pb2].at[off2]],
                                      rows[j2], gsem[j2]).wait()
                pltpu.async_copy(rows[j2], acc.at[colb[pb2].at[off2]],
                                 ssem[j2], add=True)
        for k in range(_GEPB - _GNBUF, _GEPB):
            j = k % _GNBUF
            pbp = (k // _GCHB) % 2
            pltpu.make_async_copy(rows[j], acc.at[colb[pbp].at[k % _GCHB]],
                                  ssem[j]).wait()
        plsc.subcore_barrier()
        pltpu.sync_copy(acc.at[pl.ds(base, _RPS)],
                        out_hbm.at[c, pl.ds(base, _RPS)])

    return run(hs, row2d, col2d, zrows)


def _dis_from_parts(dp):
    deg = dp[0, :, 0:1] + dp[1, :, 0:1] + 1.0
    return lax.rsqrt(deg)


def _fastkan(x, ln_g, ln_b, swr, bwT, bb):
    m = jnp.mean(x, axis=1, keepdims=True)
    v = jnp.mean((x - m) ** 2, axis=1, keepdims=True)
    xn = (x - m) * lax.rsqrt(v + 1e-5) * ln_g + ln_b
    acc = jnp.dot(jax.nn.silu(x), bwT,
                  preferred_element_type=jnp.float32) + bb
    for g0 in range(_G):
        basis = jnp.exp(-(((xn - _GRIDS[g0]) / _DENOM) ** 2))
        acc = acc + jnp.dot(basis, swr[g0],
                            preferred_element_type=jnp.float32)
    return acc


_BLK = 2000
_NSTEP = _N // _BLK


def _full(shape):
    r = len(shape)
    return pl.BlockSpec(shape, lambda i, _r=r: (0,) * _r)


def _rows(shape):
    r = len(shape)
    return pl.BlockSpec((_BLK,) + tuple(shape[1:]),
                        lambda i, _r=r: (i,) + (0,) * (_r - 1))


_DP_SPEC = pl.BlockSpec((2, _BLK, _H), lambda i: (0, i, 0))


def _tc_stage1(x, dp, ln_g, ln_b, swr, bwT, bb):
    def body(x_ref, dp_ref, g_ref, b_ref, swr_ref, bwT_ref, bb_ref, hs_ref):
        dis = _dis_from_parts(dp_ref[...])
        h = _fastkan(x_ref[...], g_ref[...], b_ref[...], swr_ref[...],
                     bwT_ref[...], bb_ref[...])
        hs_ref[...] = h * dis

    return pl.pallas_call(
        body,
        grid=(_NSTEP,),
        in_specs=[_rows(x.shape), _DP_SPEC, _full(ln_g.shape),
                  _full(ln_b.shape), _full(swr.shape), _full(bwT.shape),
                  _full(bb.shape)],
        out_specs=_rows((_N, _H)),
        out_shape=jax.ShapeDtypeStruct((_N, _H), jnp.float32),
    )(x, dp, ln_g, ln_b, swr, bwT, bb)


def _tc_aggstats(p, hs, dp, bias):
    def body(p_ref, hs_ref, dp_ref, bias_ref, agg_ref, st_ref):
        i = pl.program_id(0)
        dis = _dis_from_parts(dp_ref[...])
        agg = (p_ref[0] + p_ref[1] + hs_ref[...]) * dis + bias_ref[...]
        agg_ref[...] = agg

        @pl.when(i == 0)
        def _():
            st_ref[...] = jnp.zeros((2, _H), jnp.float32)

        st_ref[...] += jnp.stack(
            [jnp.sum(agg, axis=0), jnp.sum(agg * agg, axis=0)])

    return pl.pallas_call(
        body,
        grid=(_NSTEP,),
        in_specs=[pl.BlockSpec((2, _BLK, _H), lambda i: (0, i, 0)),
                  _rows((_N, _H)), _DP_SPEC, _full(bias.shape)],
        out_specs=[_rows((_N, _H)), _full((2, _H))],
        out_shape=[jax.ShapeDtypeStruct((_N, _H), jnp.float32),
                   jax.ShapeDtypeStruct((2, _H), jnp.float32)],
    )(p, hs, dp, bias)


def _bn_from_stats(x, st, g, b):
    m = st[0:1] / float(_N)
    v = st[1:2] / float(_N) - m * m
    return (x - m) * lax.rsqrt(v + 1e-5) * g + b


def _tc_stage2(agg, st, dp, bn_g, bn_b, ln_g, ln_b, swr, bwT, bb):
    def body(agg_ref, st_ref, dp_ref, bng_ref, bnb_ref, lng_ref, lnb_ref,
             swr_ref, bwT_ref, bb_ref, h1p_ref, hs2_ref):
        dis = _dis_from_parts(dp_ref[...])
        h1p = _bn_from_stats(agg_ref[...], st_ref[...], bng_ref[...],
                             bnb_ref[...])
        h1p_ref[...] = h1p
        h2 = _fastkan(h1p, lng_ref[...], lnb_ref[...], swr_ref[...],
                      bwT_ref[...], bb_ref[...])
        hs2_ref[...] = h2 * dis

    return pl.pallas_call(
        body,
        grid=(_NSTEP,),
        in_specs=[_rows((_N, _H)), _full((2, _H)), _DP_SPEC,
                  _full(bn_g.shape), _full(bn_b.shape), _full(ln_g.shape),
                  _full(ln_b.shape), _full(swr.shape), _full(bwT.shape),
                  _full(bb.shape)],
        out_specs=[_rows((_N, _H)), _rows((_N, _H))],
        out_shape=[jax.ShapeDtypeStruct((_N, _H), jnp.float32),
                   jax.ShapeDtypeStruct((_N, _H), jnp.float32)],
    )(agg, st, dp, bn_g, bn_b, ln_g, ln_b, swr, bwT, bb)


def _tc_stage3(x, h1p, agg2, st2, bn_g, bn_b, lng_r, lnb_r, swro, bwTo, bbo):
    def body(x_ref, h1p_ref, agg_ref, st_ref, bng_ref, bnb_ref, lng_ref,
             lnb_ref, swr_ref, bwT_ref, bb_ref, out_ref):
        h2p = _bn_from_stats(agg_ref[...], st_ref[...], bng_ref[...],
                             bnb_ref[...])
        pieces = (x_ref[...], h1p_ref[...], h2p)
        din = float(3 * _H)
        m = (sum(jnp.sum(p, axis=1, keepdims=True) for p in pieces)) / din
        ssd = sum(jnp.sum((p - m) ** 2, axis=1, keepdims=True)
                  for p in pieces)
        inv = lax.rsqrt(ssd / din + 1e-5)
        acc = jnp.zeros((_BLK, _C), jnp.float32) + bb_ref[...]
        for pi, piece in enumerate(pieces):
            xn = (piece - m) * inv * lng_ref[pi] + lnb_ref[pi]
            acc = acc + jnp.dot(jax.nn.silu(piece), bwT_ref[pi],
                                preferred_element_type=jnp.float32)
            for g0 in range(_G):
                basis = jnp.exp(-(((xn - _GRIDS[g0]) / _DENOM) ** 2))
                acc = acc + jnp.dot(basis, swr_ref[pi, g0],
                                    preferred_element_type=jnp.float32)
        out_ref[...] = acc

    return pl.pallas_call(
        body,
        grid=(_NSTEP,),
        in_specs=[_rows((_N, _F)), _rows((_N, _H)), _rows((_N, _H)),
                  _full((2, _H)), _full(bn_g.shape), _full(bn_b.shape),
                  _full(lng_r.shape), _full(lnb_r.shape), _full(swro.shape),
                  _full(bwTo.shape), _full(bbo.shape)],
        out_specs=_rows((_N, _C)),
        out_shape=jax.ShapeDtypeStruct((_N, _C), jnp.float32),
    )(x, h1p, agg2, st2, bn_g, bn_b, lng_r, lnb_r, swro, bwTo, bbo)


def kernel(x, edge_index, ln_g1, ln_b1, sw1, bw1, bb1, bias1, bn_g1, bn_b1,
           ln_g2, ln_b2, sw2, bw2, bb2, bias2, bn_g2, bn_b2, ln_go, ln_bo,
           swo, bwo, bbo):
    pad = _EPAD - _E
    pad_iota = lax.iota(jnp.int32, pad)
    rowp = jnp.concatenate([edge_index[0], pad_iota % _N])
    colp = jnp.concatenate([edge_index[1], _N + pad_iota % (_NP - _N)])
    col2d = colp.reshape(_NBP, _LANES)
    row2d64 = rowp.reshape(_NW * _GEPB, _GL)
    col2d64 = colp.reshape(_NW * _GEPB, _GL)
    zrows = jnp.zeros((_RPS, _H), jnp.float32)
    ones_rows = jnp.ones((_LANES, _H), jnp.float32)

    swr1 = jnp.transpose(sw1.reshape(_H, _F, _G), (2, 1, 0))
    swr2 = jnp.transpose(sw2.reshape(_H, _H, _G), (2, 1, 0))
    swro = jnp.transpose(swo.reshape(_C, 3, _H, _G), (1, 3, 2, 0))
    bwTo = jnp.transpose(bwo.reshape(_C, 3, _H), (1, 2, 0))

    dp = _sc_degree(col2d, ones_rows, zrows)[:, :_N]
    hs1 = _tc_stage1(x, dp, ln_g1, ln_b1, swr1, bw1.T, bb1)
    p1 = _sc_segment_sum(hs1, row2d64, col2d64, zrows)[:, :_N]
    agg1, st1 = _tc_aggstats(p1, hs1, dp, bias1)
    h1p, hs2 = _tc_stage2(agg1, st1, dp, bn_g1, bn_b1, ln_g2, ln_b2,
                          swr2, bw2.T, bb2)
    p2 = _sc_segment_sum(hs2, row2d64, col2d64, zrows)[:, :_N]
    agg2, st2 = _tc_aggstats(p2, hs2, dp, bias2)
    return _tc_stage3(x, h1p, agg2, st2, bn_g2, bn_b2,
                      ln_go.reshape(3, _H), ln_bo.reshape(3, _H), swro,
                      bwTo, bbo)


# speedup vs baseline: 2.9167x; 1.0567x over previous
import functools

import jax
import jax.numpy as jnp
from jax import lax
from jax.experimental import pallas as pl
from jax.experimental.pallas import tpu as pltpu
from jax.experimental.pallas import tpu_sc as plsc

_N = 10000
_E = 320000
_F = 128
_H = 128
_C = 40
_G = 4
_LANES = 128
_NC = 2
_NS = 16
_NW = _NC * _NS
_EPB = 80
_NBP = _NW * _EPB
_EPAD = _NBP * _LANES
_GL = 64
_GEPB = _EPAD // (_NW * _GL)
_GCHB = 32
_GNCH = _GEPB // _GCHB
_GNBUF = 4
_NP = 10240
_RPS = _NP // _NS

_GRID_MIN, _GRID_MAX = -2.0, 2.0
_DENOM = (_GRID_MAX - _GRID_MIN) / (_G - 1)
_GRIDS = tuple(_GRID_MIN + i * _DENOM for i in range(_G))


def _sc_degree(col2d, ones_rows, zrows):
    mesh = plsc.VectorSubcoreMesh(core_axis_name="c", subcore_axis_name="s")

    @functools.partial(
        pl.kernel,
        mesh=mesh,
        out_type=jax.ShapeDtypeStruct((_NC, _NP, _H), jnp.float32),
        scratch_types=[
            pltpu.VMEM((_EPB, _LANES), jnp.int32),
            pltpu.VMEM((_LANES, _H), jnp.float32),
            pltpu.VMEM_SHARED((_NP, _H), jnp.float32),
            pltpu.SemaphoreType.DMA,
        ],
    )
    def run(col_hbm, ones_hbm, z_hbm, out_hbm, colb, onesv, acc, sem):
        c = lax.axis_index("c")
        s = lax.axis_index("s")
        wid = c * _NS + s
        base = pl.multiple_of(s * _RPS, _RPS)
        bbase = pl.multiple_of(wid * _EPB, 8)
        pltpu.sync_copy(z_hbm, acc.at[pl.ds(base, _RPS)])
        pltpu.sync_copy(col_hbm.at[pl.ds(bbase, _EPB)], colb)
        pltpu.sync_copy(ones_hbm, onesv)
        plsc.subcore_barrier()

        def round_body(r, carry):
            for j in range(8):
                k = r * 8 + j
                pltpu.async_copy(onesv, acc.at[colb.at[k]], sem, add=True)
            for j in range(8):
                k = r * 8 + j
                pltpu.make_async_copy(onesv, acc.at[colb.at[k]], sem).wait()
            return carry

        lax.fori_loop(0, _EPB // 8, round_body, 0)
        plsc.subcore_barrier()
        pltpu.sync_copy(acc.at[pl.ds(base, _RPS)],
                        out_hbm.at[c, pl.ds(base, _RPS)])

    return run(col2d, ones_rows, zrows)


def _sc_segment_sum(hs, row2d, col2d, zrows):
    mesh = plsc.VectorSubcoreMesh(core_axis_name="c", subcore_axis_name="s")

    @functools.partial(
        pl.kernel,
        mesh=mesh,
        out_type=jax.ShapeDtypeStruct((_NC, _NP, _H), jnp.float32),
        scratch_types=(
            [pltpu.VMEM((_GCHB, _GL), jnp.int32)] * 4
            + [pltpu.VMEM((_GL, _H), jnp.float32)] * _GNBUF
            + [pltpu.VMEM_SHARED((_NP, _H), jnp.float32)]
            + [pltpu.SemaphoreType.DMA] * (2 * _GNBUF + 4)
        ),
    )
    def run(hs_hbm, row_hbm, col_hbm, z_hbm, out_hbm, *rest):
        rowb = rest[0:2]
        colb = rest[2:4]
        rows = rest[4:4 + _GNBUF]
        acc = rest[4 + _GNBUF]
        sems = rest[5 + _GNBUF:]
        gsem = sems[0:_GNBUF]
        ssem = sems[_GNBUF:2 * _GNBUF]
        ibr = sems[2 * _GNBUF:2 * _GNBUF + 2]
        ibc = sems[2 * _GNBUF + 2:2 * _GNBUF + 4]
        c = lax.axis_index("c")
        s = lax.axis_index("s")
        wid = c * _NS + s
        base = pl.multiple_of(s * _RPS, _RPS)
        bbase = pl.multiple_of(wid * _GEPB, 8)

        def idx_fetch(ch):
            pb = ch % 2
            cb = pl.multiple_of(bbase + ch * _GCHB, 8)
            pltpu.async_copy(row_hbm.at[pl.ds(cb, _GCHB)], rowb[pb], ibr[pb])
            pltpu.async_copy(col_hbm.at[pl.ds(cb, _GCHB)], colb[pb], ibc[pb])

        def idx_wait(ch):
            pb = ch % 2
            cb = pl.multiple_of(bbase + ch * _GCHB, 8)
            pltpu.make_async_copy(row_hbm.at[pl.ds(cb, _GCHB)], rowb[pb],
                                  ibr[pb]).wait()
            pltpu.make_async_copy(col_hbm.at[pl.ds(cb, _GCHB)], colb[pb],
                                  ibc[pb]).wait()

        pltpu.sync_copy(z_hbm, acc.at[pl.ds(base, _RPS)])
        idx_fetch(0)
        idx_fetch(1)
        plsc.subcore_barrier()

        for t in range(_GEPB + 2):
            if t < _GEPB:
                k = t
                ch, off = divmod(k, _GCHB)
                pb = ch % 2
                j = k % _GNBUF
                if off == 0:
                    idx_wait(ch)
                if off == 4 and ch + 1 < _GNCH:
                    idx_fetch(ch + 1)
                if k >= _GNBUF:
                    kp = k - _GNBUF
                    pbp = (kp // _GCHB) % 2
                    pltpu.make_async_copy(
                        rows[j], acc.at[colb[pbp].at[kp % _GCHB]],
                        ssem[j]).wait()
                pltpu.async_copy(hs_hbm.at[rowb[pb].at[off]], rows[j],
                                 gsem[j])
            if t >= 2:
                k2 = t - 2
                ch2, off2 = divmod(k2, _GCHB)
                pb2 = ch2 % 2
                j2 = k2 % _GNBUF
                pltpu.make_async_copy(hs_hbm.at[rowb[pb2].at[off2]],
                                      rows[j2], gsem[j2]).wait()
                pltpu.async_copy(rows[j2], acc.at[colb[pb2].at[off2]],
                                 ssem[j2], add=True)
        for k in range(_GEPB - _GNBUF, _GEPB):
            j = k % _GNBUF
            pbp = (k // _GCHB) % 2
            pltpu.make_async_copy(rows[j], acc.at[colb[pbp].at[k % _GCHB]],
                                  ssem[j]).wait()
        plsc.subcore_barrier()
        pltpu.sync_copy(acc.at[pl.ds(base, _RPS)],
                        out_hbm.at[c, pl.ds(base, _RPS)])

    return run(hs, row2d, col2d, zrows)


def _dis_from_parts(dp):
    deg = dp[0, :, 0:1] + dp[1, :, 0:1] + 1.0
    return lax.rsqrt(deg)


def _fastkan(x, ln_g, ln_b, swr, bwT, bb):
    m = jnp.mean(x, axis=1, keepdims=True)
    v = jnp.mean((x - m) ** 2, axis=1, keepdims=True)
    xn = (x - m) * lax.rsqrt(v + 1e-5) * ln_g + ln_b
    acc = jnp.dot(jax.nn.silu(x), bwT,
                  preferred_element_type=jnp.float32) + bb
    for g0 in range(_G):
        basis = jnp.exp(-(((xn - _GRIDS[g0]) / _DENOM) ** 2))
        acc = acc + jnp.dot(basis, swr[g0],
                            preferred_element_type=jnp.float32)
    return acc


_BLK = 2000
_NSTEP = _N // _BLK


def _full(shape):
    r = len(shape)
    return pl.BlockSpec(shape, lambda i, _r=r: (0,) * _r)


def _rows(shape):
    r = len(shape)
    return pl.BlockSpec((_BLK,) + tuple(shape[1:]),
                        lambda i, _r=r: (i,) + (0,) * (_r - 1))


_DP_SPEC = pl.BlockSpec((2, _BLK, _H), lambda i: (0, i, 0))


def _tc_stage1(x, dp, ln_g, ln_b, swr, bwT, bb):
    def body(x_ref, dp_ref, g_ref, b_ref, swr_ref, bwT_ref, bb_ref, hs_ref,
             disv_ref):
        dis = _dis_from_parts(dp_ref[...])
        h = _fastkan(x_ref[...], g_ref[...], b_ref[...], swr_ref[...],
                     bwT_ref[...], bb_ref[...])
        hs_ref[...] = h * dis
        disv_ref[...] = jnp.broadcast_to(dis, (_BLK, 8))

    return pl.pallas_call(
        body,
        grid=(_NSTEP,),
        in_specs=[_rows(x.shape), _DP_SPEC, _full(ln_g.shape),
                  _full(ln_b.shape), _full(swr.shape), _full(bwT.shape),
                  _full(bb.shape)],
        out_specs=[_rows((_N, _H)), _rows((_N, 8))],
        out_shape=[jax.ShapeDtypeStruct((_N, _H), jnp.float32),
                   jax.ShapeDtypeStruct((_N, 8), jnp.float32)],
    )(x, dp, ln_g, ln_b, swr, bwT, bb)


def _tc_aggstats(p, hs, disv, bias):
    def body(p_ref, hs_ref, disv_ref, bias_ref, agg_ref, st_ref):
        i = pl.program_id(0)
        dis = disv_ref[...][:, 0:1]
        agg = (p_ref[0] + p_ref[1] + hs_ref[...]) * dis + bias_ref[...]
        agg_ref[...] = agg

        @pl.when(i == 0)
        def _():
            st_ref[...] = jnp.zeros((2, _H), jnp.float32)

        st_ref[...] += jnp.stack(
            [jnp.sum(agg, axis=0), jnp.sum(agg * agg, axis=0)])

    return pl.pallas_call(
        body,
        grid=(_NSTEP,),
        in_specs=[pl.BlockSpec((2, _BLK, _H), lambda i: (0, i, 0)),
                  _rows((_N, _H)), _rows((_N, 8)), _full(bias.shape)],
        out_specs=[_rows((_N, _H)), _full((2, _H))],
        out_shape=[jax.ShapeDtypeStruct((_N, _H), jnp.float32),
                   jax.ShapeDtypeStruct((2, _H), jnp.float32)],
    )(p, hs, disv, bias)


def _bn_from_stats(x, st, g, b):
    m = st[0:1] / float(_N)
    v = st[1:2] / float(_N) - m * m
    return (x - m) * lax.rsqrt(v + 1e-5) * g + b


def _tc_stage2(agg, st, disv, bn_g, bn_b, ln_g, ln_b, swr, bwT, bb):
    def body(agg_ref, st_ref, disv_ref, bng_ref, bnb_ref, lng_ref, lnb_ref,
             swr_ref, bwT_ref, bb_ref, h1p_ref, hs2_ref):
        dis = disv_ref[...][:, 0:1]
        h1p = _bn_from_stats(agg_ref[...], st_ref[...], bng_ref[...],
                             bnb_ref[...])
        h1p_ref[...] = h1p
        h2 = _fastkan(h1p, lng_ref[...], lnb_ref[...], swr_ref[...],
                      bwT_ref[...], bb_ref[...])
        hs2_ref[...] = h2 * dis

    return pl.pallas_call(
        body,
        grid=(_NSTEP,),
        in_specs=[_rows((_N, _H)), _full((2, _H)), _rows((_N, 8)),
                  _full(bn_g.shape), _full(bn_b.shape), _full(ln_g.shape),
                  _full(ln_b.shape), _full(swr.shape), _full(bwT.shape),
                  _full(bb.shape)],
        out_specs=[_rows((_N, _H)), _rows((_N, _H))],
        out_shape=[jax.ShapeDtypeStruct((_N, _H), jnp.float32),
                   jax.ShapeDtypeStruct((_N, _H), jnp.float32)],
    )(agg, st, disv, bn_g, bn_b, ln_g, ln_b, swr, bwT, bb)


def _tc_stage3(x, h1p, agg2, st2, bn_g, bn_b, lng_r, lnb_r, swro, bwTo, bbo):
    def body(x_ref, h1p_ref, agg_ref, st_ref, bng_ref, bnb_ref, lng_ref,
             lnb_ref, swr_ref, bwT_ref, bb_ref, out_ref):
        h2p = _bn_from_stats(agg_ref[...], st_ref[...], bng_ref[...],
                             bnb_ref[...])
        pieces = (x_ref[...], h1p_ref[...], h2p)
        din = float(3 * _H)
        m = (sum(jnp.sum(p, axis=1, keepdims=True) for p in pieces)) / din
        ssd = sum(jnp.sum((p - m) ** 2, axis=1, keepdims=True)
                  for p in pieces)
        inv = lax.rsqrt(ssd / din + 1e-5)
        acc = jnp.zeros((_BLK, _C), jnp.float32) + bb_ref[...]
        for pi, piece in enumerate(pieces):
            xn = (piece - m) * inv * lng_ref[pi] + lnb_ref[pi]
            acc = acc + jnp.dot(jax.nn.silu(piece), bwT_ref[pi],
                                preferred_element_type=jnp.float32)
            for g0 in range(_G):
                basis = jnp.exp(-(((xn - _GRIDS[g0]) / _DENOM) ** 2))
                acc = acc + jnp.dot(basis, swr_ref[pi, g0],
                                    preferred_element_type=jnp.float32)
        out_ref[...] = acc

    return pl.pallas_call(
        body,
        grid=(_NSTEP,),
        in_specs=[_rows((_N, _F)), _rows((_N, _H)), _rows((_N, _H)),
                  _full((2, _H)), _full(bn_g.shape), _full(bn_b.shape),
                  _full(lng_r.shape), _full(lnb_r.shape), _full(swro.shape),
                  _full(bwTo.shape), _full(bbo.shape)],
        out_specs=_rows((_N, _C)),
        out_shape=jax.ShapeDtypeStruct((_N, _C), jnp.float32),
    )(x, h1p, agg2, st2, bn_g, bn_b, lng_r, lnb_r, swro, bwTo, bbo)


def kernel(x, edge_index, ln_g1, ln_b1, sw1, bw1, bb1, bias1, bn_g1, bn_b1,
           ln_g2, ln_b2, sw2, bw2, bb2, bias2, bn_g2, bn_b2, ln_go, ln_bo,
           swo, bwo, bbo):
    pad = _EPAD - _E
    pad_iota = lax.iota(jnp.int32, pad)
    rowp = jnp.concatenate([edge_index[0], pad_iota % _N])
    colp = jnp.concatenate([edge_index[1], _N + pad_iota % (_NP - _N)])
    col2d = colp.reshape(_NBP, _LANES)
    row2d64 = rowp.reshape(_NW * _GEPB, _GL)
    col2d64 = colp.reshape(_NW * _GEPB, _GL)
    zrows = jnp.zeros((_RPS, _H), jnp.float32)
    ones_rows = jnp.ones((_LANES, _H), jnp.float32)

    swr1 = jnp.transpose(sw1.reshape(_H, _F, _G), (2, 1, 0))
    swr2 = jnp.transpose(sw2.reshape(_H, _H, _G), (2, 1, 0))
    swro = jnp.transpose(swo.reshape(_C, 3, _H, _G), (1, 3, 2, 0))
    bwTo = jnp.transpose(bwo.reshape(_C, 3, _H), (1, 2, 0))

    dp = _sc_degree(col2d, ones_rows, zrows)
    hs1, disv = _tc_stage1(x, dp, ln_g1, ln_b1, swr1, bw1.T, bb1)
    p1 = _sc_segment_sum(hs1, row2d64, col2d64, zrows)
    agg1, st1 = _tc_aggstats(p1, hs1, disv, bias1)
    h1p, hs2 = _tc_stage2(agg1, st1, disv, bn_g1, bn_b1, ln_g2, ln_b2,
                          swr2, bw2.T, bb2)
    p2 = _sc_segment_sum(hs2, row2d64, col2d64, zrows)
    agg2, st2 = _tc_aggstats(p2, hs2, disv, bias2)
    return _tc_stage3(x, h1p, agg2, st2, bn_g2, bn_b2,
                      ln_go.reshape(3, _H), ln_bo.reshape(3, _H), swro,
                      bwTo, bbo)


# speedup vs baseline: 2.9360x; 1.0066x over previous
import functools

import jax
import jax.numpy as jnp
from jax import lax
from jax.experimental import pallas as pl
from jax.experimental.pallas import tpu as pltpu
from jax.experimental.pallas import tpu_sc as plsc

_N = 10000
_E = 320000
_F = 128
_H = 128
_C = 40
_G = 4
_LANES = 128
_NC = 2
_NS = 16
_NW = _NC * _NS
_EPB = 80
_NBP = _NW * _EPB
_EPAD = _NBP * _LANES
_GL = 64
_GEPB = _EPAD // (_NW * _GL)
_GCHB = 32
_GNCH = _GEPB // _GCHB
_GNBUF = 4
_NP = 10240
_RPS = _NP // _NS

_GRID_MIN, _GRID_MAX = -2.0, 2.0
_DENOM = (_GRID_MAX - _GRID_MIN) / (_G - 1)
_GRIDS = tuple(_GRID_MIN + i * _DENOM for i in range(_G))


def _sc_degree(col2d, ones_rows, zrows):
    mesh = plsc.VectorSubcoreMesh(core_axis_name="c", subcore_axis_name="s")

    @functools.partial(
        pl.kernel,
        mesh=mesh,
        out_type=jax.ShapeDtypeStruct((_NC, _NP, _H), jnp.float32),
        scratch_types=[
            pltpu.VMEM((_EPB, _LANES), jnp.int32),
            pltpu.VMEM((_LANES, _H), jnp.float32),
            pltpu.VMEM_SHARED((_NP, _H), jnp.float32),
            pltpu.SemaphoreType.DMA,
        ],
    )
    def run(col_hbm, ones_hbm, z_hbm, out_hbm, colb, onesv, acc, sem):
        c = lax.axis_index("c")
        s = lax.axis_index("s")
        wid = c * _NS + s
        base = pl.multiple_of(s * _RPS, _RPS)
        bbase = pl.multiple_of(wid * _EPB, 8)
        pltpu.sync_copy(z_hbm, acc.at[pl.ds(base, _RPS)])
        pltpu.sync_copy(col_hbm.at[pl.ds(bbase, _EPB)], colb)
        pltpu.sync_copy(ones_hbm, onesv)
        plsc.subcore_barrier()

        def round_body(r, carry):
            for j in range(8):
                k = r * 8 + j
                pltpu.async_copy(onesv, acc.at[colb.at[k]], sem, add=True)
            for j in range(8):
                k = r * 8 + j
                pltpu.make_async_copy(onesv, acc.at[colb.at[k]], sem).wait()
            return carry

        lax.fori_loop(0, _EPB // 8, round_body, 0)
        plsc.subcore_barrier()
        pltpu.sync_copy(acc.at[pl.ds(base, _RPS)],
                        out_hbm.at[c, pl.ds(base, _RPS)])

    return run(col2d, ones_rows, zrows)


def _sc_segment_sum(hs, row2d, col2d, zrows):
    mesh = plsc.VectorSubcoreMesh(core_axis_name="c", subcore_axis_name="s")

    @functools.partial(
        pl.kernel,
        mesh=mesh,
        out_type=jax.ShapeDtypeStruct((_NC, _NP, _H), jnp.float32),
        scratch_types=(
            [pltpu.VMEM((_GCHB, _GL), jnp.int32)] * 4
            + [pltpu.VMEM((_GL, _H), jnp.float32)] * _GNBUF
            + [pltpu.VMEM_SHARED((_NP, _H), jnp.float32)]
            + [pltpu.SemaphoreType.DMA] * (2 * _GNBUF + 4)
        ),
    )
    def run(hs_hbm, row_hbm, col_hbm, z_hbm, out_hbm, *rest):
        rowb = rest[0:2]
        colb = rest[2:4]
        rows = rest[4:4 + _GNBUF]
        acc = rest[4 + _GNBUF]
        sems = rest[5 + _GNBUF:]
        gsem = sems[0:_GNBUF]
        ssem = sems[_GNBUF:2 * _GNBUF]
        ibr = sems[2 * _GNBUF:2 * _GNBUF + 2]
        ibc = sems[2 * _GNBUF + 2:2 * _GNBUF + 4]
        c = lax.axis_index("c")
        s = lax.axis_index("s")
        wid = c * _NS + s
        base = pl.multiple_of(s * _RPS, _RPS)
        bbase = pl.multiple_of(wid * _GEPB, 8)

        def idx_fetch(ch):
            pb = ch % 2
            cb = pl.multiple_of(bbase + ch * _GCHB, 8)
            pltpu.async_copy(row_hbm.at[pl.ds(cb, _GCHB)], rowb[pb], ibr[pb])
            pltpu.async_copy(col_hbm.at[pl.ds(cb, _GCHB)], colb[pb], ibc[pb])

        def idx_wait(ch):
            pb = ch % 2
            cb = pl.multiple_of(bbase + ch * _GCHB, 8)
            pltpu.make_async_copy(row_hbm.at[pl.ds(cb, _GCHB)], rowb[pb],
                                  ibr[pb]).wait()
            pltpu.make_async_copy(col_hbm.at[pl.ds(cb, _GCHB)], colb[pb],
                                  ibc[pb]).wait()

        pltpu.sync_copy(z_hbm, acc.at[pl.ds(base, _RPS)])
        idx_fetch(0)
        idx_fetch(1)
        plsc.subcore_barrier()

        for t in range(_GEPB + 2):
            if t < _GEPB:
                k = t
                ch, off = divmod(k, _GCHB)
                pb = ch % 2
                j = k % _GNBUF
                if off == 0:
                    idx_wait(ch)
                if off == 4 and ch + 1 < _GNCH:
                    idx_fetch(ch + 1)
                if k >= _GNBUF:
                    kp = k - _GNBUF
                    pbp = (kp // _GCHB) % 2
                    pltpu.make_async_copy(
                        rows[j], acc.at[colb[pbp].at[kp % _GCHB]],
                        ssem[j]).wait()
                pltpu.async_copy(hs_hbm.at[rowb[pb].at[off]], rows[j],
                                 gsem[j])
            if t >= 2:
                k2 = t - 2
                ch2, off2 = divmod(k2, _GCHB)
                pb2 = ch2 % 2
                j2 = k2 % _GNBUF
                pltpu.make_async_copy(hs_hbm.at[rowb[pb2].at[off2]],
                                      rows[j2], gsem[j2]).wait()
                pltpu.async_copy(rows[j2], acc.at[colb[pb2].at[off2]],
                                 ssem[j2], add=True)
        for k in range(_GEPB - _GNBUF, _GEPB):
            j = k % _GNBUF
            pbp = (k // _GCHB) % 2
            pltpu.make_async_copy(rows[j], acc.at[colb[pbp].at[k % _GCHB]],
                                  ssem[j]).wait()
        plsc.subcore_barrier()
        pltpu.sync_copy(acc.at[pl.ds(base, _RPS)],
                        out_hbm.at[c, pl.ds(base, _RPS)])

    return run(hs, row2d, col2d, zrows)


def _dis_from_parts(dp):
    deg = dp[0, :, 0:1] + dp[1, :, 0:1] + 1.0
    return lax.rsqrt(deg)


def _fastkan(x, ln_g, ln_b, swr, bwT, bb):
    m = jnp.mean(x, axis=1, keepdims=True)
    v = jnp.mean((x - m) ** 2, axis=1, keepdims=True)
    xn = (x - m) * lax.rsqrt(v + 1e-5) * ln_g + ln_b
    acc = jnp.dot(jax.nn.silu(x), bwT,
                  preferred_element_type=jnp.float32) + bb
    for g0 in range(_G):
        basis = jnp.exp(-(((xn - _GRIDS[g0]) / _DENOM) ** 2))
        acc = acc + jnp.dot(basis, swr[g0],
                            preferred_element_type=jnp.float32)
    return acc


_BLK = 2000
_NSTEP = _N // _BLK


def _full(shape):
    r = len(shape)
    return pl.BlockSpec(shape, lambda i, _r=r: (0,) * _r)


def _rows(shape):
    r = len(shape)
    return pl.BlockSpec((_BLK,) + tuple(shape[1:]),
                        lambda i, _r=r: (i,) + (0,) * (_r - 1))


_DP_SPEC = pl.BlockSpec((2, _BLK, _H), lambda i: (0, i, 0))


def _tc_fastkan1(x, ln_g, ln_b, swr, bwT, bb):
    def body(x_ref, g_ref, b_ref, swr_ref, bwT_ref, bb_ref, h_ref):
        h_ref[...] = _fastkan(x_ref[...], g_ref[...], b_ref[...],
                              swr_ref[...], bwT_ref[...], bb_ref[...])

    return pl.pallas_call(
        body,
        grid=(_NSTEP,),
        in_specs=[_rows(x.shape), _full(ln_g.shape), _full(ln_b.shape),
                  _full(swr.shape), _full(bwT.shape), _full(bb.shape)],
        out_specs=_rows((_N, _H)),
        out_shape=jax.ShapeDtypeStruct((_N, _H), jnp.float32),
    )(x, ln_g, ln_b, swr, bwT, bb)


def _tc_scale(h, dp):
    def body(h_ref, dp_ref, hs_ref, disv_ref):
        dis = _dis_from_parts(dp_ref[...])
        hs_ref[...] = h_ref[...] * dis
        disv_ref[...] = jnp.broadcast_to(dis, (_BLK, 8))

    return pl.pallas_call(
        body,
        grid=(_NSTEP,),
        in_specs=[_rows((_N, _H)), _DP_SPEC],
        out_specs=[_rows((_N, _H)), _rows((_N, 8))],
        out_shape=[jax.ShapeDtypeStruct((_N, _H), jnp.float32),
                   jax.ShapeDtypeStruct((_N, 8), jnp.float32)],
    )(h, dp)


def _tc_aggstats(p, hs, disv, bias):
    def body(p_ref, hs_ref, disv_ref, bias_ref, agg_ref, st_ref):
        i = pl.program_id(0)
        dis = disv_ref[...][:, 0:1]
        agg = (p_ref[0] + p_ref[1] + hs_ref[...]) * dis + bias_ref[...]
        agg_ref[...] = agg

        @pl.when(i == 0)
        def _():
            st_ref[...] = jnp.zeros((2, _H), jnp.float32)

        st_ref[...] += jnp.stack(
            [jnp.sum(agg, axis=0), jnp.sum(agg * agg, axis=0)])

    return pl.pallas_call(
        body,
        grid=(_NSTEP,),
        in_specs=[pl.BlockSpec((2, _BLK, _H), lambda i: (0, i, 0)),
                  _rows((_N, _H)), _rows((_N, 8)), _full(bias.shape)],
        out_specs=[_rows((_N, _H)), _full((2, _H))],
        out_shape=[jax.ShapeDtypeStruct((_N, _H), jnp.float32),
                   jax.ShapeDtypeStruct((2, _H), jnp.float32)],
    )(p, hs, disv, bias)


def _bn_from_stats(x, st, g, b):
    m = st[0:1] / float(_N)
    v = st[1:2] / float(_N) - m * m
    return (x - m) * lax.rsqrt(v + 1e-5) * g + b


def _tc_stage2(agg, st, disv, bn_g, bn_b, ln_g, ln_b, swr, bwT, bb):
    def body(agg_ref, st_ref, disv_ref, bng_ref, bnb_ref, lng_ref, lnb_ref,
             swr_ref, bwT_ref, bb_ref, h1p_ref, hs2_ref):
        dis = disv_ref[...][:, 0:1]
        h1p = _bn_from_stats(agg_ref[...], st_ref[...], bng_ref[...],
                             bnb_ref[...])
        h1p_ref[...] = h1p
        h2 = _fastkan(h1p, lng_ref[...], lnb_ref[...], swr_ref[...],
                      bwT_ref[...], bb_ref[...])
        hs2_ref[...] = h2 * dis

    return pl.pallas_call(
        body,
        grid=(_NSTEP,),
        in_specs=[_rows((_N, _H)), _full((2, _H)), _rows((_N, 8)),
                  _full(bn_g.shape), _full(bn_b.shape), _full(ln_g.shape),
                  _full(ln_b.shape), _full(swr.shape), _full(bwT.shape),
                  _full(bb.shape)],
        out_specs=[_rows((_N, _H)), _rows((_N, _H))],
        out_shape=[jax.ShapeDtypeStruct((_N, _H), jnp.float32),
                   jax.ShapeDtypeStruct((_N, _H), jnp.float32)],
    )(agg, st, disv, bn_g, bn_b, ln_g, ln_b, swr, bwT, bb)


def _tc_stage3(x, h1p, agg2, st2, bn_g, bn_b, lng_r, lnb_r, swro, bwTo, bbo):
    def body(x_ref, h1p_ref, agg_ref, st_ref, bng_ref, bnb_ref, lng_ref,
             lnb_ref, swr_ref, bwT_ref, bb_ref, out_ref):
        h2p = _bn_from_stats(agg_ref[...], st_ref[...], bng_ref[...],
                             bnb_ref[...])
        pieces = (x_ref[...], h1p_ref[...], h2p)
        din = float(3 * _H)
        m = (sum(jnp.sum(p, axis=1, keepdims=True) for p in pieces)) / din
        ssd = sum(jnp.sum((p - m) ** 2, axis=1, keepdims=True)
                  for p in pieces)
        inv = lax.rsqrt(ssd / din + 1e-5)
        acc = jnp.zeros((_BLK, _C), jnp.float32) + bb_ref[...]
        for pi, piece in enumerate(pieces):
            xn = (piece - m) * inv * lng_ref[pi] + lnb_ref[pi]
            acc = acc + jnp.dot(jax.nn.silu(piece), bwT_ref[pi],
                                preferred_element_type=jnp.float32)
            for g0 in range(_G):
                basis = jnp.exp(-(((xn - _GRIDS[g0]) / _DENOM) ** 2))
                acc = acc + jnp.dot(basis, swr_ref[pi, g0],
                                    preferred_element_type=jnp.float32)
        out_ref[...] = acc

    return pl.pallas_call(
        body,
        grid=(_NSTEP,),
        in_specs=[_rows((_N, _F)), _rows((_N, _H)), _rows((_N, _H)),
                  _full((2, _H)), _full(bn_g.shape), _full(bn_b.shape),
                  _full(lng_r.shape), _full(lnb_r.shape), _full(swro.shape),
                  _full(bwTo.shape), _full(bbo.shape)],
        out_specs=_rows((_N, _C)),
        out_shape=jax.ShapeDtypeStruct((_N, _C), jnp.float32),
    )(x, h1p, agg2, st2, bn_g, bn_b, lng_r, lnb_r, swro, bwTo, bbo)


def kernel(x, edge_index, ln_g1, ln_b1, sw1, bw1, bb1, bias1, bn_g1, bn_b1,
           ln_g2, ln_b2, sw2, bw2, bb2, bias2, bn_g2, bn_b2, ln_go, ln_bo,
           swo, bwo, bbo):
    pad = _EPAD - _E
    pad_iota = lax.iota(jnp.int32, pad)
    rowp = jnp.concatenate([edge_index[0], pad_iota % _N])
    colp = jnp.concatenate([edge_index[1], _N + pad_iota % (_NP - _N)])
    col2d = colp.reshape(_NBP, _LANES)
    row2d64 = rowp.reshape(_NW * _GEPB, _GL)
    col2d64 = colp.reshape(_NW * _GEPB, _GL)
    zrows = jnp.zeros((_RPS, _H), jnp.float32)
    ones_rows = jnp.ones((_LANES, _H), jnp.float32)

    swr1 = jnp.transpose(sw1.reshape(_H, _F, _G), (2, 1, 0))
    swr2 = jnp.transpose(sw2.reshape(_H, _H, _G), (2, 1, 0))
    swro = jnp.transpose(swo.reshape(_C, 3, _H, _G), (1, 3, 2, 0))
    bwTo = jnp.transpose(bwo.reshape(_C, 3, _H), (1, 2, 0))

    dp = _sc_degree(col2d, ones_rows, zrows)
    h1 = _tc_fastkan1(x, ln_g1, ln_b1, swr1, bw1.T, bb1)
    hs1, disv = _tc_scale(h1, dp)
    p1 = _sc_segment_sum(hs1, row2d64, col2d64, zrows)
    agg1, st1 = _tc_aggstats(p1, hs1, disv, bias1)
    h1p, hs2 = _tc_stage2(agg1, st1, disv, bn_g1, bn_b1, ln_g2, ln_b2,
                          swr2, bw2.T, bb2)
    p2 = _sc_segment_sum(hs2, row2d64, col2d64, zrows)
    agg2, st2 = _tc_aggstats(p2, hs2, disv, bias2)
    return _tc_stage3(x, h1p, agg2, st2, bn_g2, bn_b2,
                      ln_go.reshape(3, _H), ln_bo.reshape(3, _H), swro,
                      bwTo, bbo)
